# baseline (speedup 1.0000x reference)
import dataclasses
import functools

import jax
import jax.numpy as jnp
from jax import lax
from jax.experimental import pallas as pl
from jax.experimental.pallas import tpu as pltpu
from jax.experimental.pallas import tpu_sc as plsc

H = 128
LANES = 16
NW = 32
F32 = jnp.float32
I32 = jnp.int32


def _mesh():
    return plsc.VectorSubcoreMesh(core_axis_name="c", subcore_axis_name="s")


def _sc_params():
    cp = pltpu.CompilerParams()
    if "needs_layout_passes" in pltpu.CompilerParams.__dataclass_fields__:
        cp = dataclasses.replace(cp, needs_layout_passes=False)
    return cp


def _wid():
    return lax.axis_index("s") * 2 + lax.axis_index("c")


def _sload(vref, j):
    g = (j // LANES) * LANES
    grp = vref[pl.ds(g, LANES)]
    lane = lax.broadcasted_iota(I32, (LANES,), 0)
    return jnp.sum(jnp.where(lane == (j - g), grp, 0))


def _mm_body(a_ref, w_ref, b_ref, o_ref):
    o_ref[...] = (
        jnp.dot(a_ref[...], w_ref[...], preferred_element_type=F32, precision=lax.Precision.HIGHEST) + b_ref[...]
    )


def _mm(a, w, b, rb):
    n, k = a.shape
    hout = w.shape[1]
    return pl.pallas_call(
        _mm_body,
        grid=(n // rb,),
        in_specs=[
            pl.BlockSpec((rb, k), lambda i: (i, 0)),
            pl.BlockSpec((k, hout), lambda i: (0, 0)),
            pl.BlockSpec((1, hout), lambda i: (0, 0)),
        ],
        out_specs=pl.BlockSpec((rb, hout), lambda i: (i, 0)),
        out_shape=jax.ShapeDtypeStruct((n, hout), F32),
    )(a, w, b)


def _upd_body(attr_ref, g1_ref, g2_ref, g3_ref, wa_ref, wb_ref, wc_ref, o_ref):
    v = attr_ref[...]
    v += jnp.maximum(jnp.dot(g1_ref[...], wa_ref[...], preferred_element_type=F32, precision=lax.Precision.HIGHEST), 0.0)
    v += jnp.maximum(jnp.dot(g2_ref[...], wb_ref[...], preferred_element_type=F32, precision=lax.Precision.HIGHEST), 0.0)
    v += jnp.maximum(jnp.dot(g3_ref[...], wc_ref[...], preferred_element_type=F32, precision=lax.Precision.HIGHEST), 0.0)
    m = jnp.mean(v, axis=-1, keepdims=True)
    var = jnp.mean((v - m) * (v - m), axis=-1, keepdims=True)
    o_ref[...] = (v - m) * lax.rsqrt(var + 1e-5)


def _upd(attr, g1, g2, g3, wa, wb, wc, rb=4000):
    n = attr.shape[0]
    bs = pl.BlockSpec((rb, H), lambda i: (i, 0))
    ws = pl.BlockSpec((H, H), lambda i: (0, 0))
    return pl.pallas_call(
        _upd_body,
        grid=(n // rb,),
        in_specs=[bs, bs, bs, bs, ws, ws, ws],
        out_specs=bs,
        out_shape=jax.ShapeDtypeStruct((n, H), F32),
    )(attr, g1, g2, g3, wa, wb, wc)


def _elu(v):
    return jnp.where(v > 0.0, v, jnp.exp(jnp.minimum(v, 0.0)) - 1.0)


def _head_body(nblk, a0_ref, n1_ref, n2_ref, b_ref, wp_ref, wm1_ref, bm1_ref,
               wm2_ref, bm2_ref, o_ref, acc, acc2):
    i = pl.program_id(0)

    @pl.when(i == 0)
    def _():
        acc[...] = jnp.zeros_like(acc)
        acc2[...] = jnp.zeros_like(acc2)

    node = a0_ref[...] + n1_ref[...] + n2_ref[...]
    grp = b_ref[0, 0, :]
    oh = (grp[:, None] == lax.broadcasted_iota(I32, (grp.shape[0], 512), 1)).astype(F32)
    dn = (((0,), (0,)), ((), ()))
    acc[...] += lax.dot_general(oh, node, dn, preferred_element_type=F32, precision=lax.Precision.HIGHEST)
    acc2[...] += lax.dot_general(oh, jnp.ones((grp.shape[0], 8), F32), dn,
                                 preferred_element_type=F32, precision=lax.Precision.HIGHEST)

    @pl.when(i == nblk - 1)
    def _():
        c = jnp.maximum(acc2[:, 0:1], 1.0)
        g = acc[...] / c
        g = _elu(jnp.dot(g, wp_ref[...], preferred_element_type=F32, precision=lax.Precision.HIGHEST))
        h1 = _elu(jnp.dot(g, wm1_ref[...], preferred_element_type=F32, precision=lax.Precision.HIGHEST) + bm1_ref[...])
        res = jnp.dot(h1, wm2_ref[...], preferred_element_type=F32, precision=lax.Precision.HIGHEST) + bm2_ref[...]
        o_ref[...] = res[:, 0]


def _head(a0, n1, n2, batch_r, wp, wm1p, bm1p, wm2p, bm2p, rb=128):
    n = a0.shape[0]
    nblk = n // rb
    bs = pl.BlockSpec((rb, H), lambda i: (i, 0))
    return pl.pallas_call(
        functools.partial(_head_body, nblk),
        grid=(nblk,),
        in_specs=[
            bs, bs, bs,
            pl.BlockSpec((1, 1, rb), lambda i: (i, 0, 0)),
            pl.BlockSpec((H, H), lambda i: (0, 0)),
            pl.BlockSpec((H, H), lambda i: (0, 0)),
            pl.BlockSpec((1, H), lambda i: (0, 0)),
            pl.BlockSpec((H, H), lambda i: (0, 0)),
            pl.BlockSpec((1, H), lambda i: (0, 0)),
        ],
        out_specs=pl.BlockSpec((512,), lambda i: (0,)),
        out_shape=jax.ShapeDtypeStruct((512,), F32),
        scratch_shapes=[pltpu.VMEM((512, H), F32), pltpu.VMEM((512, 8), F32)],
    )(a0, n1, n2, batch_r, wp, wm1p, bm1p, wm2p, bm2p)


def _edge_init(base, table, aidx, bidx, c1=200):
    e = aidx.shape[0]
    per_w = e // NW
    nchunks = per_w // c1
    has_base = base is not None

    scratch = [
        pltpu.VMEM((c1,), I32),
        pltpu.VMEM((c1,), I32),
        pltpu.VMEM((c1, H), F32),
        pltpu.VMEM((c1, H), F32),
    ]
    if has_base:
        scratch.append(pltpu.VMEM((c1, H), F32))

    def body(*refs):
        if has_base:
            base_h, tab_h, a_h, b_h, o_h, av, bv, g1, g2, ob = refs
        else:
            tab_h, a_h, b_h, o_h, av, bv, g1, g2 = refs
            ob = g1
        w = _wid()
        row0 = w * per_w

        @pl.loop(0, nchunks)
        def _(ci):
            st = row0 + ci * c1
            pltpu.sync_copy(a_h.at[pl.ds(st, c1)], av)
            pltpu.sync_copy(b_h.at[pl.ds(st, c1)], bv)
            pltpu.sync_copy(tab_h.at[av], g1)
            pltpu.sync_copy(tab_h.at[bv], g2)
            if has_base:
                pltpu.sync_copy(base_h.at[pl.ds(st, c1)], ob)

            @plsc.parallel_loop(0, c1, 1, unroll=4)
            def _(i):
                for k in range(H // LANES):
                    s = pl.ds(k * LANES, LANES)
                    if has_base:
                        ob[i, s] = ob[i, s] + g1[i, s] + g2[i, s]
                    else:
                        ob[i, s] = g1[i, s] + g2[i, s]

            pltpu.sync_copy(ob, o_h.at[pl.ds(st, c1)])

    kern = pl.kernel(
        body,
        out_type=jax.ShapeDtypeStruct((e, H), F32),
        mesh=_mesh(),
        scratch_types=scratch,
        name="sc_edge_init",
    )
    if has_base:
        return kern(base, table, aidx, bidx)
    return kern(table, aidx, bidx)


def _gmix(tab0, x, e0, e1, inv, c1=200):
    e = e0.shape[0]
    per_w = e // NW
    nchunks = per_w // c1

    def body(t0_h, x_h, e0_h, e1_h, iv_h, o_h, av, bv, cv, ga, gb, gi, xl):
        w = _wid()
        row0 = w * per_w

        @pl.loop(0, nchunks)
        def _(ci):
            st = row0 + ci * c1
            pltpu.sync_copy(e0_h.at[pl.ds(st, c1)], av)
            pltpu.sync_copy(e1_h.at[pl.ds(st, c1)], bv)
            pltpu.sync_copy(iv_h.at[pl.ds(st, c1)], cv)
            pltpu.sync_copy(t0_h.at[av], ga)
            pltpu.sync_copy(t0_h.at[bv], gb)
            pltpu.sync_copy(x_h.at[cv], gi)
            pltpu.sync_copy(x_h.at[pl.ds(st, c1)], xl)

            @plsc.parallel_loop(0, c1, 1, unroll=4)
            def _(i):
                for k in range(H // LANES):
                    s = pl.ds(k * LANES, LANES)
                    xl[i, s] = ga[i, s] * xl[i, s] + gb[i, s] * gi[i, s]

            pltpu.sync_copy(xl, o_h.at[pl.ds(st, c1)])

    return pl.kernel(
        body,
        out_type=jax.ShapeDtypeStruct((e, H), F32),
        mesh=_mesh(),
        scratch_types=[
            pltpu.VMEM((c1,), I32),
            pltpu.VMEM((c1,), I32),
            pltpu.VMEM((c1,), I32),
            pltpu.VMEM((c1, H), F32),
            pltpu.VMEM((c1, H), F32),
            pltpu.VMEM((c1, H), F32),
            pltpu.VMEM((c1, H), F32),
        ],
        name="sc_gmix",
    )(tab0, x, e0, e1, inv)


def _sym(x, inv, c1=200):
    e = x.shape[0]
    per_w = e // NW
    nchunks = per_w // c1

    def body(x_h, iv_h, o_h, cv, gi, xl):
        w = _wid()
        row0 = w * per_w

        @pl.loop(0, nchunks)
        def _(ci):
            st = row0 + ci * c1
            pltpu.sync_copy(iv_h.at[pl.ds(st, c1)], cv)
            pltpu.sync_copy(x_h.at[cv], gi)
            pltpu.sync_copy(x_h.at[pl.ds(st, c1)], xl)

            @plsc.parallel_loop(0, c1, 1, unroll=4)
            def _(i):
                for k in range(H // LANES):
                    s = pl.ds(k * LANES, LANES)
                    xl[i, s] = 0.5 * (xl[i, s] + gi[i, s])

            pltpu.sync_copy(xl, o_h.at[pl.ds(st, c1)])

    return pl.kernel(
        body,
        out_type=jax.ShapeDtypeStruct((e, H), F32),
        mesh=_mesh(),
        scratch_types=[
            pltpu.VMEM((c1,), I32),
            pltpu.VMEM((c1, H), F32),
            pltpu.VMEM((c1, H), F32),
        ],
        name="sc_sym",
    )(x, inv)


def _tri_agg(table, a_s, b_s, t_s, offs, nout, rng, c2, pair):
    nr = nout // rng
    offpad = offs.shape[0]
    reg = rng + 8

    scratch = [
        pltpu.VMEM((offpad,), I32),
        pltpu.VMEM((c2,), I32),
        pltpu.VMEM((c2,), I32),
        pltpu.VMEM((c2,), I32),
        pltpu.VMEM((c2, H), F32),
        pltpu.VMEM((rng, H), F32),
        pltpu.VMEM_SHARED((16 * reg, H), F32),
    ]
    if pair:
        scratch.insert(2, pltpu.VMEM((c2,), I32))
        scratch.append(pltpu.VMEM((c2, H), F32))

    def body(*refs):
        if pair:
            (tab_h, a_h, b_h, t_h, off_h, o_h,
             offv, av, bv, tv, tlv, g1, zb, acc, g2) = refs
        else:
            tab_h, a_h, t_h, off_h, o_h, offv, av, tv, tlv, g1, zb, acc = refs
        w = _wid()
        sid = lax.axis_index("s")
        base = sid * reg
        pltpu.sync_copy(off_h, offv)

        @plsc.parallel_loop(0, rng, 1, unroll=4)
        def _(i):
            for k in range(H // LANES):
                zb[i, pl.ds(k * LANES, LANES)] = jnp.zeros((LANES,), F32)

        nmine = (nr - w + NW - 1) // NW

        @pl.loop(0, nmine)
        def _(rc):
            r = w + rc * NW
            o0 = _sload(offv, r)
            o1 = _sload(offv, r + 1)
            start = (o0 // 8) * 8
            nch = (o1 - start + c2 - 1) // c2
            lo = r * rng
            pltpu.sync_copy(zb, acc.at[pl.ds(base, rng)])

            @pl.loop(0, nch)
            def _(ci):
                st = start + ci * c2
                pltpu.sync_copy(a_h.at[pl.ds(st, c2)], av)
                pltpu.sync_copy(t_h.at[pl.ds(st, c2)], tv)
                pltpu.sync_copy(tab_h.at[av], g1)
                if pair:
                    pltpu.sync_copy(b_h.at[pl.ds(st, c2)], bv)
                    pltpu.sync_copy(tab_h.at[bv], g2)

                @plsc.parallel_loop(0, c2 // LANES, 1, unroll=4)
                def _(gi):
                    s = pl.ds(gi * LANES, LANES)
                    t16 = tv[s]
                    inr = (t16 >= lo) & (t16 < lo + rng)
                    tlv[s] = jnp.where(inr, t16 - lo + base, base + rng)

                if pair:
                    @plsc.parallel_loop(0, c2, 1, unroll=4)
                    def _(i):
                        for k in range(H // LANES):
                            s = pl.ds(k * LANES, LANES)
                            g1[i, s] = g1[i, s] * g2[i, s]

                pltpu.sync_copy(g1, acc.at[tlv], add=True)

            pltpu.sync_copy(acc.at[pl.ds(base, rng)], o_h.at[pl.ds(lo, rng)])

    kern = pl.kernel(
        body,
        out_type=jax.ShapeDtypeStruct((nout, H), F32),
        mesh=_mesh(),
        scratch_types=scratch,
        compiler_params=_sc_params(),
        name="sc_tri_agg" if pair else "sc_seg_add",
    )
    if pair:
        return kern(table, a_s, b_s, t_s, offs)
    return kern(table, a_s, t_s, offs)


def _mkoffs(t_s, nout, rng):
    nr = nout // rng
    offs = jnp.searchsorted(
        t_s, (jnp.arange(nr + 1, dtype=I32) * rng).astype(I32)
    ).astype(I32)
    offpad = ((nr + 1 + 15) // 16) * 16
    return jnp.pad(offs, (0, offpad - nr - 1), constant_values=t_s.shape[0])


def _sort3(tgt, a, b, nout, rng, c2):
    t_s, a_s, b_s = lax.sort([tgt, a, b], num_keys=1)
    offs = _mkoffs(t_s, nout, rng)
    pad = c2 + 8
    t_s = jnp.pad(t_s, (0, pad), constant_values=1 << 30)
    a_s = jnp.pad(a_s, (0, pad))
    b_s = jnp.pad(b_s, (0, pad))
    return t_s, a_s, b_s, offs


def _sort2(tgt, a, nout, rng, c2):
    t_s, a_s = lax.sort([tgt, a], num_keys=1)
    offs = _mkoffs(t_s, nout, rng)
    pad = c2 + 8
    t_s = jnp.pad(t_s, (0, pad), constant_values=1 << 30)
    a_s = jnp.pad(a_s, (0, pad))
    return t_s, a_s, offs


def kernel(x, edge_attr, edge_index, edge_index2, triangle_1_1_1,
           triangle_1_1_2, triangle_2_2_1, triangle_2_2_2, inverse_edge_1,
           inverse_edge_2, num_nodes, batch0, W0, b0, W1, b1, W2, b2, Wk, Wp,
           Wm1, bm1, Wm2, bm2):
    n = x.shape[0]
    e1 = edge_index.shape[1]
    e2 = edge_index2.shape[1]
    nl = Wk.shape[0]
    npad = 10240

    i32 = lambda v: v.astype(I32)
    ei0, ei1 = i32(edge_index[0]), i32(edge_index[1])
    e20, e21 = i32(edge_index2[0]), i32(edge_index2[1])
    inv1, inv2 = i32(inverse_edge_1), i32(inverse_edge_2)

    RNG, C2 = 128, 256
    s111 = _sort3(i32(triangle_1_1_1[0]), i32(triangle_1_1_1[1]),
                  i32(triangle_1_1_1[2]), e1, RNG, C2)
    s112 = _sort3(i32(triangle_1_1_2[2]), i32(triangle_1_1_2[0]),
                  i32(triangle_1_1_2[1]), e2, RNG, C2)
    s221 = _sort3(i32(triangle_2_2_1[2]), i32(triangle_2_2_1[0]),
                  i32(triangle_2_2_1[1]), e1, RNG, C2)
    s222 = _sort3(i32(triangle_2_2_2[2]), i32(triangle_2_2_2[0]),
                  i32(triangle_2_2_2[1]), e2, RNG, C2)

    NRNG, NC2 = 32, 512
    ar1 = jnp.arange(e1, dtype=I32)
    n1s = _sort2(jnp.concatenate([ei0, ei1]), jnp.concatenate([ar1, ar1]),
                 npad, NRNG, NC2)
    ar2 = jnp.arange(e2, dtype=I32)
    n2s = _sort2(jnp.concatenate([e20, e21]), jnp.concatenate([ar2, ar2]),
                 npad, NRNG, NC2)

    xpad = jnp.zeros((npad, 16), F32).at[:n, :11].set(x)
    wcat = jnp.zeros((16, 3 * H), F32).at[:11].set(
        jnp.concatenate([W0, W1[5:], W2], axis=1)
    )
    bcat = jnp.concatenate([b0, jnp.zeros((H,), F32), 0.5 * b2])[None]
    xt = _mm(xpad, wcat, bcat, rb=2048)
    attr0 = xt[:, :H]
    xw1h = xt[:, H:2 * H]
    xw2b = xt[:, 2 * H:]

    epad = jnp.zeros((e1, 8), F32).at[:, :5].set(edge_attr)
    w1a = jnp.zeros((8, H), F32).at[:5].set(W1[:5])
    edgepart = _mm(epad, w1a, b1[None], rb=4000)

    attr1 = _edge_init(edgepart, xw1h, ei0, ei1)
    attr2 = _edge_init(None, xw2b, e20, e21)

    for l in range(nl):
        agg111 = _tri_agg(attr1, s111[1], s111[2], s111[0], s111[3], e1, RNG, C2, True)
        agg112 = _tri_agg(attr1, s112[1], s112[2], s112[0], s112[3], e2, RNG, C2, True)
        agg221 = _tri_agg(attr2, s221[1], s221[2], s221[0], s221[3], e1, RNG, C2, True)
        agg222 = _tri_agg(attr2, s222[1], s222[2], s222[0], s222[3], e2, RNG, C2, True)
        agg011 = _gmix(attr0, attr1, ei0, ei1, inv1)
        agg022 = _gmix(attr0, attr2, e20, e21, inv2)
        attr1 = _upd(attr1, agg111, agg011, agg221, Wk[l, 0], Wk[l, 1], Wk[l, 4])
        attr2 = _upd(attr2, agg022, agg112, agg222, Wk[l, 2], Wk[l, 3], Wk[l, 5])
        attr1 = _sym(attr1, inv1)
        attr2 = _sym(attr2, inv2)

    na1 = _tri_agg(attr1, n1s[1], None, n1s[0], n1s[2], npad, NRNG, NC2, False)
    na2 = _tri_agg(attr2, n2s[1], None, n2s[0], n2s[2], npad, NRNG, NC2, False)

    bpad = jnp.full((npad,), 600, I32).at[:n].set(i32(batch0))
    batch_r = bpad.reshape(npad // 128, 1, 128)
    wm1p = jnp.zeros((H, H), F32).at[:, :Wm1.shape[1]].set(Wm1)
    bm1p = jnp.zeros((1, H), F32).at[0, :bm1.shape[0]].set(bm1)
    wm2p = jnp.zeros((H, H), F32).at[:Wm2.shape[0], 0].set(Wm2[:, 0])
    bm2p = jnp.zeros((1, H), F32) + bm2[0]
    out = _head(attr0, na1, na2, batch_r, Wp, wm1p, bm1p, wm2p, bm2p)
    return out


# speedup vs baseline: 1.1825x; 1.1825x over previous
import dataclasses
import functools

import jax
import jax.numpy as jnp
from jax import lax
from jax.experimental import pallas as pl
from jax.experimental.pallas import tpu as pltpu
from jax.experimental.pallas import tpu_sc as plsc

H = 128
LANES = 16
NW = 32
F32 = jnp.float32
I32 = jnp.int32


def _mesh():
    return plsc.VectorSubcoreMesh(core_axis_name="c", subcore_axis_name="s")


def _sc_params():
    cp = pltpu.CompilerParams()
    if "needs_layout_passes" in pltpu.CompilerParams.__dataclass_fields__:
        cp = dataclasses.replace(cp, needs_layout_passes=False)
    return cp


def _wid():
    return lax.axis_index("s") * 2 + lax.axis_index("c")


def _sload(vref, j):
    g = (j // LANES) * LANES
    grp = vref[pl.ds(g, LANES)]
    lane = lax.broadcasted_iota(I32, (LANES,), 0)
    return jnp.sum(jnp.where(lane == (j - g), grp, 0))


def _mm_body(a_ref, w_ref, b_ref, o_ref):
    o_ref[...] = (
        jnp.dot(a_ref[...], w_ref[...], preferred_element_type=F32, precision=lax.Precision.HIGHEST) + b_ref[...]
    )


def _mm(a, w, b, rb):
    n, k = a.shape
    hout = w.shape[1]
    return pl.pallas_call(
        _mm_body,
        grid=(n // rb,),
        in_specs=[
            pl.BlockSpec((rb, k), lambda i: (i, 0)),
            pl.BlockSpec((k, hout), lambda i: (0, 0)),
            pl.BlockSpec((1, hout), lambda i: (0, 0)),
        ],
        out_specs=pl.BlockSpec((rb, hout), lambda i: (i, 0)),
        out_shape=jax.ShapeDtypeStruct((n, hout), F32),
    )(a, w, b)


def _upd_body(attr_ref, g1_ref, g2_ref, g3_ref, wa_ref, wb_ref, wc_ref, o_ref):
    v = attr_ref[...]
    v += jnp.maximum(jnp.dot(g1_ref[...], wa_ref[...], preferred_element_type=F32, precision=lax.Precision.HIGHEST), 0.0)
    v += jnp.maximum(jnp.dot(g2_ref[...], wb_ref[...], preferred_element_type=F32, precision=lax.Precision.HIGHEST), 0.0)
    v += jnp.maximum(jnp.dot(g3_ref[...], wc_ref[...], preferred_element_type=F32, precision=lax.Precision.HIGHEST), 0.0)
    m = jnp.mean(v, axis=-1, keepdims=True)
    var = jnp.mean((v - m) * (v - m), axis=-1, keepdims=True)
    o_ref[...] = (v - m) * lax.rsqrt(var + 1e-5)


def _upd(attr, g1, g2, g3, wa, wb, wc, rb=4000):
    n = attr.shape[0]
    bs = pl.BlockSpec((rb, H), lambda i: (i, 0))
    ws = pl.BlockSpec((H, H), lambda i: (0, 0))
    return pl.pallas_call(
        _upd_body,
        grid=(n // rb,),
        in_specs=[bs, bs, bs, bs, ws, ws, ws],
        out_specs=bs,
        out_shape=jax.ShapeDtypeStruct((n, H), F32),
    )(attr, g1, g2, g3, wa, wb, wc)


def _elu(v):
    return jnp.where(v > 0.0, v, jnp.exp(jnp.minimum(v, 0.0)) - 1.0)


def _head_body(nblk, a0_ref, n1_ref, n2_ref, b_ref, wp_ref, wm1_ref, bm1_ref,
               wm2_ref, bm2_ref, o_ref, acc, acc2):
    i = pl.program_id(0)

    @pl.when(i == 0)
    def _():
        acc[...] = jnp.zeros_like(acc)
        acc2[...] = jnp.zeros_like(acc2)

    node = a0_ref[...] + n1_ref[...] + n2_ref[...]
    grp = b_ref[0, 0, :]
    oh = (grp[:, None] == lax.broadcasted_iota(I32, (grp.shape[0], 512), 1)).astype(F32)
    dn = (((0,), (0,)), ((), ()))
    acc[...] += lax.dot_general(oh, node, dn, preferred_element_type=F32, precision=lax.Precision.HIGHEST)
    acc2[...] += lax.dot_general(oh, jnp.ones((grp.shape[0], 8), F32), dn,
                                 preferred_element_type=F32, precision=lax.Precision.HIGHEST)

    @pl.when(i == nblk - 1)
    def _():
        c = jnp.maximum(acc2[:, 0:1], 1.0)
        g = acc[...] / c
        g = _elu(jnp.dot(g, wp_ref[...], preferred_element_type=F32, precision=lax.Precision.HIGHEST))
        h1 = _elu(jnp.dot(g, wm1_ref[...], preferred_element_type=F32, precision=lax.Precision.HIGHEST) + bm1_ref[...])
        res = jnp.dot(h1, wm2_ref[...], preferred_element_type=F32, precision=lax.Precision.HIGHEST) + bm2_ref[...]
        o_ref[...] = res[:, 0]


def _head(a0, n1, n2, batch_r, wp, wm1p, bm1p, wm2p, bm2p, rb=128):
    n = a0.shape[0]
    nblk = n // rb
    bs = pl.BlockSpec((rb, H), lambda i: (i, 0))
    return pl.pallas_call(
        functools.partial(_head_body, nblk),
        grid=(nblk,),
        in_specs=[
            bs, bs, bs,
            pl.BlockSpec((1, 1, rb), lambda i: (i, 0, 0)),
            pl.BlockSpec((H, H), lambda i: (0, 0)),
            pl.BlockSpec((H, H), lambda i: (0, 0)),
            pl.BlockSpec((1, H), lambda i: (0, 0)),
            pl.BlockSpec((H, H), lambda i: (0, 0)),
            pl.BlockSpec((1, H), lambda i: (0, 0)),
        ],
        out_specs=pl.BlockSpec((512,), lambda i: (0,)),
        out_shape=jax.ShapeDtypeStruct((512,), F32),
        scratch_shapes=[pltpu.VMEM((512, H), F32), pltpu.VMEM((512, 8), F32)],
    )(a0, n1, n2, batch_r, wp, wm1p, bm1p, wm2p, bm2p)


def _edge_init(base, table, aidx, bidx, c1=200):
    e = aidx.shape[0]
    per_w = e // NW
    nchunks = per_w // c1
    has_base = base is not None

    scratch = [
        pltpu.VMEM((c1,), I32),
        pltpu.VMEM((c1,), I32),
        pltpu.VMEM((c1, H), F32),
        pltpu.VMEM((c1, H), F32),
    ]
    if has_base:
        scratch.append(pltpu.VMEM((c1, H), F32))

    def body(*refs):
        if has_base:
            base_h, tab_h, a_h, b_h, o_h, av, bv, g1, g2, ob = refs
        else:
            tab_h, a_h, b_h, o_h, av, bv, g1, g2 = refs
            ob = g1
        w = _wid()
        row0 = w * per_w

        @pl.loop(0, nchunks)
        def _(ci):
            st = row0 + ci * c1
            pltpu.sync_copy(a_h.at[pl.ds(st, c1)], av)
            pltpu.sync_copy(b_h.at[pl.ds(st, c1)], bv)
            pltpu.sync_copy(tab_h.at[av], g1)
            pltpu.sync_copy(tab_h.at[bv], g2)
            if has_base:
                pltpu.sync_copy(base_h.at[pl.ds(st, c1)], ob)

            @plsc.parallel_loop(0, c1, 1, unroll=4)
            def _(i):
                for k in range(H // LANES):
                    s = pl.ds(k * LANES, LANES)
                    if has_base:
                        ob[i, s] = ob[i, s] + g1[i, s] + g2[i, s]
                    else:
                        ob[i, s] = g1[i, s] + g2[i, s]

            pltpu.sync_copy(ob, o_h.at[pl.ds(st, c1)])

    kern = pl.kernel(
        body,
        out_type=jax.ShapeDtypeStruct((e, H), F32),
        mesh=_mesh(),
        scratch_types=scratch,
        name="sc_edge_init",
    )
    if has_base:
        return kern(base, table, aidx, bidx)
    return kern(table, aidx, bidx)


def _gmix(tab0, x, e0, e1, inv, c1=200):
    e = e0.shape[0]
    per_w = e // NW
    nchunks = per_w // c1

    def body(t0_h, x_h, e0_h, e1_h, iv_h, o_h, av, bv, cv, ga, gb, gi, xl):
        w = _wid()
        row0 = w * per_w

        @pl.loop(0, nchunks)
        def _(ci):
            st = row0 + ci * c1
            pltpu.sync_copy(e0_h.at[pl.ds(st, c1)], av)
            pltpu.sync_copy(e1_h.at[pl.ds(st, c1)], bv)
            pltpu.sync_copy(iv_h.at[pl.ds(st, c1)], cv)
            pltpu.sync_copy(t0_h.at[av], ga)
            pltpu.sync_copy(t0_h.at[bv], gb)
            pltpu.sync_copy(x_h.at[cv], gi)
            pltpu.sync_copy(x_h.at[pl.ds(st, c1)], xl)

            @plsc.parallel_loop(0, c1, 1, unroll=4)
            def _(i):
                for k in range(H // LANES):
                    s = pl.ds(k * LANES, LANES)
                    xl[i, s] = ga[i, s] * xl[i, s] + gb[i, s] * gi[i, s]

            pltpu.sync_copy(xl, o_h.at[pl.ds(st, c1)])

    return pl.kernel(
        body,
        out_type=jax.ShapeDtypeStruct((e, H), F32),
        mesh=_mesh(),
        scratch_types=[
            pltpu.VMEM((c1,), I32),
            pltpu.VMEM((c1,), I32),
            pltpu.VMEM((c1,), I32),
            pltpu.VMEM((c1, H), F32),
            pltpu.VMEM((c1, H), F32),
            pltpu.VMEM((c1, H), F32),
            pltpu.VMEM((c1, H), F32),
        ],
        name="sc_gmix",
    )(tab0, x, e0, e1, inv)


def _sym(x, inv, c1=200):
    e = x.shape[0]
    per_w = e // NW
    nchunks = per_w // c1

    def body(x_h, iv_h, o_h, cv, gi, xl):
        w = _wid()
        row0 = w * per_w

        @pl.loop(0, nchunks)
        def _(ci):
            st = row0 + ci * c1
            pltpu.sync_copy(iv_h.at[pl.ds(st, c1)], cv)
            pltpu.sync_copy(x_h.at[cv], gi)
            pltpu.sync_copy(x_h.at[pl.ds(st, c1)], xl)

            @plsc.parallel_loop(0, c1, 1, unroll=4)
            def _(i):
                for k in range(H // LANES):
                    s = pl.ds(k * LANES, LANES)
                    xl[i, s] = 0.5 * (xl[i, s] + gi[i, s])

            pltpu.sync_copy(xl, o_h.at[pl.ds(st, c1)])

    return pl.kernel(
        body,
        out_type=jax.ShapeDtypeStruct((e, H), F32),
        mesh=_mesh(),
        scratch_types=[
            pltpu.VMEM((c1,), I32),
            pltpu.VMEM((c1, H), F32),
            pltpu.VMEM((c1, H), F32),
        ],
        name="sc_sym",
    )(x, inv)


def _tri_agg(table, a_s, b_s, t_s, offs, nout, rng, c2, pair):
    nr = nout // rng
    nmine = nr // NW
    assert nr % NW == 0 and nmine % 2 == 0
    offpad = offs.shape[0]
    reg = rng + 8

    idx_t = lambda: pltpu.VMEM((c2,), I32)
    g_t = lambda: pltpu.VMEM((c2, H), F32)
    if pair:
        scratch = [pltpu.VMEM((offpad,), I32), pltpu.VMEM((rng, H), F32),
                   idx_t(),
                   idx_t(), idx_t(), idx_t(),
                   idx_t(), idx_t(), idx_t(),
                   idx_t(), idx_t(), idx_t(),
                   g_t(), g_t(), g_t(), g_t(),
                   pltpu.VMEM_SHARED((16 * reg, H), F32),
                   pltpu.SemaphoreType.DMA, pltpu.SemaphoreType.DMA,
                   pltpu.SemaphoreType.DMA, pltpu.SemaphoreType.DMA]
    else:
        scratch = [pltpu.VMEM((offpad,), I32), pltpu.VMEM((rng, H), F32),
                   idx_t(),
                   idx_t(), idx_t(),
                   idx_t(), idx_t(),
                   idx_t(), idx_t(),
                   g_t(), g_t(),
                   pltpu.VMEM_SHARED((16 * reg, H), F32),
                   pltpu.SemaphoreType.DMA, pltpu.SemaphoreType.DMA,
                   pltpu.SemaphoreType.DMA, pltpu.SemaphoreType.DMA]

    def body(*refs):
        if pair:
            (tab_h, a_h, b_h, t_h, off_h, o_h,
             offv, zb, tlv, avA, tvA, bvA, avB, tvB, bvB, av2, tv2, bv2,
             g1A, g2A, g1B, g2B, acc, siA, siB, sgA, sgB) = refs
        else:
            (tab_h, a_h, t_h, off_h, o_h,
             offv, zb, tlv, avA, tvA, avB, tvB, av2, tv2,
             g1A, g1B, acc, siA, siB, sgA, sgB) = refs
            bvA = bvB = bv2 = g2A = g2B = None
        w = _wid()
        sid = lax.axis_index("s")
        base = sid * reg
        pltpu.sync_copy(off_h, offv)

        @plsc.parallel_loop(0, rng, 1, unroll=4)
        def _(i):
            for k in range(H // LANES):
                zb[i, pl.ds(k * LANES, LANES)] = jnp.zeros((LANES,), F32)

        pltpu.sync_copy(zb, acc.at[pl.ds(base, rng)])

        def startof(k):
            return (_sload(offv, w + k * NW) // 8) * 8

        def issue_idx(k, av, tv, bv, sem):
            st = startof(k)
            pltpu.async_copy(a_h.at[pl.ds(st, c2)], av, sem)
            pltpu.async_copy(t_h.at[pl.ds(st, c2)], tv, sem)
            if pair:
                pltpu.async_copy(b_h.at[pl.ds(st, c2)], bv, sem)

        def wait_idx(av, tv, bv, sem):
            pltpu.make_async_copy(a_h.at[pl.ds(0, c2)], av, sem).wait()
            pltpu.make_async_copy(t_h.at[pl.ds(0, c2)], tv, sem).wait()
            if pair:
                pltpu.make_async_copy(b_h.at[pl.ds(0, c2)], bv, sem).wait()

        def issue_gather(av, bv, g1, g2, sem):
            pltpu.async_copy(tab_h.at[av], g1, sem)
            if pair:
                pltpu.async_copy(tab_h.at[bv], g2, sem)

        def wait_gather(av, bv, g1, g2, sem):
            pltpu.make_async_copy(tab_h.at[av], g1, sem).wait()
            if pair:
                pltpu.make_async_copy(tab_h.at[bv], g2, sem).wait()

        def accumulate(tv_, g1_, g2_, lo):
            @plsc.parallel_loop(0, c2 // LANES, 1, unroll=4)
            def _(gi):
                sl = pl.ds(gi * LANES, LANES)
                t16 = tv_[sl]
                inr = (t16 >= lo) & (t16 < lo + rng)
                tlv[sl] = jnp.where(inr, t16 - lo + base, base + rng)

            if pair:
                @plsc.parallel_loop(0, c2, 1, unroll=4)
                def _(i):
                    for k in range(H // LANES):
                        sl = pl.ds(k * LANES, LANES)
                        g1_[i, sl] = g1_[i, sl] * g2_[i, sl]

            pltpu.sync_copy(g1_, acc.at[tlv], add=True)

        def process(k, tvP, g1P, g2P):
            r = w + k * NW
            o0 = _sload(offv, r)
            o1 = _sload(offv, r + 1)
            start = (o0 // 8) * 8
            lo = r * rng
            accumulate(tvP, g1P, g2P, lo)
            nch = (o1 - start + c2 - 1) // c2

            @pl.loop(1, nch)
            def _(ci):
                st = start + ci * c2
                pltpu.sync_copy(a_h.at[pl.ds(st, c2)], av2)
                pltpu.sync_copy(t_h.at[pl.ds(st, c2)], tv2)
                if pair:
                    pltpu.sync_copy(b_h.at[pl.ds(st, c2)], bv2)
                pltpu.sync_copy(tab_h.at[av2], g1P)
                if pair:
                    pltpu.sync_copy(tab_h.at[bv2], g2P)
                accumulate(tv2, g1P, g2P, lo)

            pltpu.sync_copy(acc.at[pl.ds(base, rng)], o_h.at[pl.ds(lo, rng)])
            pltpu.sync_copy(zb, acc.at[pl.ds(base, rng)])

        issue_idx(0, avA, tvA, bvA, siA)
        wait_idx(avA, tvA, bvA, siA)
        issue_gather(avA, bvA, g1A, g2A, sgA)
        issue_idx(1, avB, tvB, bvB, siB)

        def phase(k, avP, tvP, bvP, g1P, g2P, siP, sgP,
                  avQ, tvQ, bvQ, g1Q, g2Q, siQ, sgQ):
            wait_gather(avP, bvP, g1P, g2P, sgP)

            @pl.when(k + 1 < nmine)
            def _():
                wait_idx(avQ, tvQ, bvQ, siQ)
                issue_gather(avQ, bvQ, g1Q, g2Q, sgQ)

                @pl.when(k + 2 < nmine)
                def _():
                    issue_idx(k + 2, avP, tvP, bvP, siP)

            process(k, tvP, g1P, g2P)

        @pl.loop(0, nmine // 2)
        def _(k2):
            k = 2 * k2
            phase(k, avA, tvA, bvA, g1A, g2A, siA, sgA,
                  avB, tvB, bvB, g1B, g2B, siB, sgB)
            phase(k + 1, avB, tvB, bvB, g1B, g2B, siB, sgB,
                  avA, tvA, bvA, g1A, g2A, siA, sgA)

    kern = pl.kernel(
        body,
        out_type=jax.ShapeDtypeStruct((nout, H), F32),
        mesh=_mesh(),
        scratch_types=scratch,
        compiler_params=_sc_params(),
        name="sc_tri_agg" if pair else "sc_seg_add",
    )
    if pair:
        return kern(table, a_s, b_s, t_s, offs)
    return kern(table, a_s, t_s, offs)


def _mkoffs(t_s, nout, rng):
    nr = nout // rng
    offs = jnp.searchsorted(
        t_s, (jnp.arange(nr + 1, dtype=I32) * rng).astype(I32)
    ).astype(I32)
    offpad = ((nr + 1 + 15) // 16) * 16
    return jnp.pad(offs, (0, offpad - nr - 1), constant_values=t_s.shape[0])


def _sort3(tgt, a, b, nout, rng, c2):
    t_s, a_s, b_s = lax.sort([tgt, a, b], num_keys=1)
    offs = _mkoffs(t_s, nout, rng)
    pad = c2 + 8
    t_s = jnp.pad(t_s, (0, pad), constant_values=1 << 30)
    a_s = jnp.pad(a_s, (0, pad))
    b_s = jnp.pad(b_s, (0, pad))
    return t_s, a_s, b_s, offs


def _sort2(tgt, a, nout, rng, c2):
    t_s, a_s = lax.sort([tgt, a], num_keys=1)
    offs = _mkoffs(t_s, nout, rng)
    pad = c2 + 8
    t_s = jnp.pad(t_s, (0, pad), constant_values=1 << 30)
    a_s = jnp.pad(a_s, (0, pad))
    return t_s, a_s, offs


def kernel(x, edge_attr, edge_index, edge_index2, triangle_1_1_1,
           triangle_1_1_2, triangle_2_2_1, triangle_2_2_2, inverse_edge_1,
           inverse_edge_2, num_nodes, batch0, W0, b0, W1, b1, W2, b2, Wk, Wp,
           Wm1, bm1, Wm2, bm2):
    n = x.shape[0]
    e1 = edge_index.shape[1]
    e2 = edge_index2.shape[1]
    nl = Wk.shape[0]
    npad = 10240

    i32 = lambda v: v.astype(I32)
    ei0, ei1 = i32(edge_index[0]), i32(edge_index[1])
    e20, e21 = i32(edge_index2[0]), i32(edge_index2[1])
    inv1, inv2 = i32(inverse_edge_1), i32(inverse_edge_2)

    RNG, C2 = 40, 128
    s111 = _sort3(i32(triangle_1_1_1[0]), i32(triangle_1_1_1[1]),
                  i32(triangle_1_1_1[2]), e1, RNG, C2)
    s112 = _sort3(i32(triangle_1_1_2[2]), i32(triangle_1_1_2[0]),
                  i32(triangle_1_1_2[1]), e2, RNG, C2)
    s221 = _sort3(i32(triangle_2_2_1[2]), i32(triangle_2_2_1[0]),
                  i32(triangle_2_2_1[1]), e1, RNG, C2)
    s222 = _sort3(i32(triangle_2_2_2[2]), i32(triangle_2_2_2[0]),
                  i32(triangle_2_2_2[1]), e2, RNG, C2)

    NRNG, NC2 = 32, 256
    ar1 = jnp.arange(e1, dtype=I32)
    n1s = _sort2(jnp.concatenate([ei0, ei1]), jnp.concatenate([ar1, ar1]),
                 npad, NRNG, NC2)
    ar2 = jnp.arange(e2, dtype=I32)
    n2s = _sort2(jnp.concatenate([e20, e21]), jnp.concatenate([ar2, ar2]),
                 npad, NRNG, NC2)

    xpad = jnp.zeros((npad, 16), F32).at[:n, :11].set(x)
    wcat = jnp.zeros((16, 3 * H), F32).at[:11].set(
        jnp.concatenate([W0, W1[5:], W2], axis=1)
    )
    bcat = jnp.concatenate([b0, jnp.zeros((H,), F32), 0.5 * b2])[None]
    xt = _mm(xpad, wcat, bcat, rb=2048)
    attr0 = xt[:, :H]
    xw1h = xt[:, H:2 * H]
    xw2b = xt[:, 2 * H:]

    epad = jnp.zeros((e1, 8), F32).at[:, :5].set(edge_attr)
    w1a = jnp.zeros((8, H), F32).at[:5].set(W1[:5])
    edgepart = _mm(epad, w1a, b1[None], rb=4000)

    attr1 = _edge_init(edgepart, xw1h, ei0, ei1)
    attr2 = _edge_init(None, xw2b, e20, e21)

    for l in range(nl):
        agg111 = _tri_agg(attr1, s111[1], s111[2], s111[0], s111[3], e1, RNG, C2, True)
        agg112 = _tri_agg(attr1, s112[1], s112[2], s112[0], s112[3], e2, RNG, C2, True)
        agg221 = _tri_agg(attr2, s221[1], s221[2], s221[0], s221[3], e1, RNG, C2, True)
        agg222 = _tri_agg(attr2, s222[1], s222[2], s222[0], s222[3], e2, RNG, C2, True)
        agg011 = _gmix(attr0, attr1, ei0, ei1, inv1)
        agg022 = _gmix(attr0, attr2, e20, e21, inv2)
        attr1 = _upd(attr1, agg111, agg011, agg221, Wk[l, 0], Wk[l, 1], Wk[l, 4])
        attr2 = _upd(attr2, agg022, agg112, agg222, Wk[l, 2], Wk[l, 3], Wk[l, 5])
        attr1 = _sym(attr1, inv1)
        attr2 = _sym(attr2, inv2)

    na1 = _tri_agg(attr1, n1s[1], None, n1s[0], n1s[2], npad, NRNG, NC2, False)
    na2 = _tri_agg(attr2, n2s[1], None, n2s[0], n2s[2], npad, NRNG, NC2, False)

    bpad = jnp.full((npad,), 600, I32).at[:n].set(i32(batch0))
    batch_r = bpad.reshape(npad // 128, 1, 128)
    wm1p = jnp.zeros((H, H), F32).at[:, :Wm1.shape[1]].set(Wm1)
    bm1p = jnp.zeros((1, H), F32).at[0, :bm1.shape[0]].set(bm1)
    wm2p = jnp.zeros((H, H), F32).at[:Wm2.shape[0], 0].set(Wm2[:, 0])
    bm2p = jnp.zeros((1, H), F32) + bm2[0]
    out = _head(attr0, na1, na2, batch_r, Wp, wm1p, bm1p, wm2p, bm2p)
    return out


# speedup vs baseline: 1.2538x; 1.0603x over previous
import dataclasses
import functools

import jax
import jax.numpy as jnp
from jax import lax
from jax.experimental import pallas as pl
from jax.experimental.pallas import tpu as pltpu
from jax.experimental.pallas import tpu_sc as plsc

H = 128
LANES = 16
NW = 32
F32 = jnp.float32
I32 = jnp.int32


def _mesh():
    return plsc.VectorSubcoreMesh(core_axis_name="c", subcore_axis_name="s")


def _sc_params():
    cp = pltpu.CompilerParams()
    if "needs_layout_passes" in pltpu.CompilerParams.__dataclass_fields__:
        cp = dataclasses.replace(cp, needs_layout_passes=False)
    return cp


def _wid():
    return lax.axis_index("s") * 2 + lax.axis_index("c")


def _sload(vref, j):
    g = (j // LANES) * LANES
    grp = vref[pl.ds(g, LANES)]
    lane = lax.broadcasted_iota(I32, (LANES,), 0)
    return jnp.sum(jnp.where(lane == (j - g), grp, 0))


def _mm_body(a_ref, w_ref, b_ref, o_ref):
    o_ref[...] = (
        jnp.dot(a_ref[...], w_ref[...], preferred_element_type=F32, precision=lax.Precision.HIGHEST) + b_ref[...]
    )


def _mm(a, w, b, rb):
    n, k = a.shape
    hout = w.shape[1]
    return pl.pallas_call(
        _mm_body,
        grid=(n // rb,),
        in_specs=[
            pl.BlockSpec((rb, k), lambda i: (i, 0)),
            pl.BlockSpec((k, hout), lambda i: (0, 0)),
            pl.BlockSpec((1, hout), lambda i: (0, 0)),
        ],
        out_specs=pl.BlockSpec((rb, hout), lambda i: (i, 0)),
        out_shape=jax.ShapeDtypeStruct((n, hout), F32),
    )(a, w, b)


def _upd_body(attr_ref, g1_ref, g2_ref, g3_ref, wa_ref, wb_ref, wc_ref, o_ref):
    v = attr_ref[...]
    v += jnp.maximum(jnp.dot(g1_ref[...], wa_ref[...], preferred_element_type=F32, precision=lax.Precision.HIGHEST), 0.0)
    v += jnp.maximum(jnp.dot(g2_ref[...], wb_ref[...], preferred_element_type=F32, precision=lax.Precision.HIGHEST), 0.0)
    v += jnp.maximum(jnp.dot(g3_ref[...], wc_ref[...], preferred_element_type=F32, precision=lax.Precision.HIGHEST), 0.0)
    m = jnp.mean(v, axis=-1, keepdims=True)
    var = jnp.mean((v - m) * (v - m), axis=-1, keepdims=True)
    o_ref[...] = (v - m) * lax.rsqrt(var + 1e-5)


def _upd(attr, g1, g2, g3, wa, wb, wc, rb=4000):
    n = attr.shape[0]
    bs = pl.BlockSpec((rb, H), lambda i: (i, 0))
    ws = pl.BlockSpec((H, H), lambda i: (0, 0))
    return pl.pallas_call(
        _upd_body,
        grid=(n // rb,),
        in_specs=[bs, bs, bs, bs, ws, ws, ws],
        out_specs=bs,
        out_shape=jax.ShapeDtypeStruct((n, H), F32),
    )(attr, g1, g2, g3, wa, wb, wc)


def _elu(v):
    return jnp.where(v > 0.0, v, jnp.exp(jnp.minimum(v, 0.0)) - 1.0)


def _head_body(nblk, a0_ref, n1_ref, n2_ref, b_ref, wp_ref, wm1_ref, bm1_ref,
               wm2_ref, bm2_ref, o_ref, acc, acc2):
    i = pl.program_id(0)

    @pl.when(i == 0)
    def _():
        acc[...] = jnp.zeros_like(acc)
        acc2[...] = jnp.zeros_like(acc2)

    node = a0_ref[...] + n1_ref[...] + n2_ref[...]
    grp = b_ref[0, 0, :]
    oh = (grp[:, None] == lax.broadcasted_iota(I32, (grp.shape[0], 512), 1)).astype(F32)
    dn = (((0,), (0,)), ((), ()))
    acc[...] += lax.dot_general(oh, node, dn, preferred_element_type=F32, precision=lax.Precision.HIGHEST)
    acc2[...] += lax.dot_general(oh, jnp.ones((grp.shape[0], 8), F32), dn,
                                 preferred_element_type=F32, precision=lax.Precision.HIGHEST)

    @pl.when(i == nblk - 1)
    def _():
        c = jnp.maximum(acc2[:, 0:1], 1.0)
        g = acc[...] / c
        g = _elu(jnp.dot(g, wp_ref[...], preferred_element_type=F32, precision=lax.Precision.HIGHEST))
        h1 = _elu(jnp.dot(g, wm1_ref[...], preferred_element_type=F32, precision=lax.Precision.HIGHEST) + bm1_ref[...])
        res = jnp.dot(h1, wm2_ref[...], preferred_element_type=F32, precision=lax.Precision.HIGHEST) + bm2_ref[...]
        o_ref[...] = res[:, 0]


def _head(a0, n1, n2, batch_r, wp, wm1p, bm1p, wm2p, bm2p, rb=128):
    n = a0.shape[0]
    nblk = n // rb
    bs = pl.BlockSpec((rb, H), lambda i: (i, 0))
    return pl.pallas_call(
        functools.partial(_head_body, nblk),
        grid=(nblk,),
        in_specs=[
            bs, bs, bs,
            pl.BlockSpec((1, 1, rb), lambda i: (i, 0, 0)),
            pl.BlockSpec((H, H), lambda i: (0, 0)),
            pl.BlockSpec((H, H), lambda i: (0, 0)),
            pl.BlockSpec((1, H), lambda i: (0, 0)),
            pl.BlockSpec((H, H), lambda i: (0, 0)),
            pl.BlockSpec((1, H), lambda i: (0, 0)),
        ],
        out_specs=pl.BlockSpec((512,), lambda i: (0,)),
        out_shape=jax.ShapeDtypeStruct((512,), F32),
        scratch_shapes=[pltpu.VMEM((512, H), F32), pltpu.VMEM((512, 8), F32)],
    )(a0, n1, n2, batch_r, wp, wm1p, bm1p, wm2p, bm2p)


def _edge_map(gathers, linears, combine, result, name, c1=40):
    e = gathers[0][1].shape[0]
    per_w = e // NW
    n = per_w // c1
    assert n % 2 == 0 and per_w % c1 == 0
    ng, nl = len(gathers), len(linears)

    scratch = []
    for _ in range(ng):
        scratch += [pltpu.VMEM((c1,), I32), pltpu.VMEM((c1,), I32)]
    for _ in range(ng + nl):
        scratch += [pltpu.VMEM((c1, H), F32), pltpu.VMEM((c1, H), F32)]
    scratch += [pltpu.SemaphoreType.DMA] * 4

    def body(*refs):
        tabs = refs[0:ng]
        idxs = refs[ng:2 * ng]
        lins = refs[2 * ng:2 * ng + nl]
        o_h = refs[2 * ng + nl]
        sc = refs[2 * ng + nl + 1:]
        ibufs = [(sc[2 * i], sc[2 * i + 1]) for i in range(ng)]
        o2 = 2 * ng
        gbufs = [(sc[o2 + 2 * i], sc[o2 + 2 * i + 1]) for i in range(ng)]
        o3 = 4 * ng
        lbufs = [(sc[o3 + 2 * j], sc[o3 + 2 * j + 1]) for j in range(nl)]
        siA, siB, sdA, sdB = sc[4 * ng + 2 * nl:4 * ng + 2 * nl + 4]
        w = _wid()
        row0 = w * per_w

        def st_of(c):
            return row0 + c * c1

        def issue_idx(c, pp, sem):
            for i in range(ng):
                pltpu.async_copy(idxs[i].at[pl.ds(st_of(c), c1)], ibufs[i][pp], sem)

        def wait_idx(pp, sem):
            for i in range(ng):
                pltpu.make_async_copy(idxs[i].at[pl.ds(0, c1)], ibufs[i][pp], sem).wait()

        def issue_data(c, pp, sem):
            for i in range(ng):
                pltpu.async_copy(tabs[i].at[ibufs[i][pp]], gbufs[i][pp], sem)
            for j in range(nl):
                pltpu.async_copy(lins[j].at[pl.ds(st_of(c), c1)], lbufs[j][pp], sem)

        def wait_data(pp, sem):
            for i in range(ng):
                pltpu.make_async_copy(tabs[i].at[ibufs[i][pp]], gbufs[i][pp], sem).wait()
            for j in range(nl):
                pltpu.make_async_copy(lins[j].at[pl.ds(0, c1)], lbufs[j][pp], sem).wait()

        issue_idx(0, 0, siA)
        wait_idx(0, siA)
        issue_data(0, 0, sdA)
        issue_idx(1, 1, siB)

        def phase(c, pp, siP, sdP, siQ, sdQ):
            qq = 1 - pp
            wait_data(pp, sdP)

            @pl.when(c + 1 < n)
            def _():
                wait_idx(qq, siQ)
                issue_data(c + 1, qq, sdQ)

                @pl.when(c + 2 < n)
                def _():
                    issue_idx(c + 2, pp, siP)

            gb = [g[pp] for g in gbufs]
            lb = [l[pp] for l in lbufs]

            @plsc.parallel_loop(0, c1, 1, unroll=4)
            def _(i):
                for k in range(H // LANES):
                    combine(gb, lb, i, pl.ds(k * LANES, LANES))

            res = gb[0] if result == "g" else lb[0]
            pltpu.sync_copy(res, o_h.at[pl.ds(st_of(c), c1)])

        @pl.loop(0, n // 2)
        def _(k2):
            c = 2 * k2
            phase(c, 0, siA, sdA, siB, sdB)
            phase(c + 1, 1, siB, sdB, siA, sdA)

    inputs = [t for t, _ in gathers] + [ix for _, ix in gathers] + list(linears)
    return pl.kernel(
        body,
        out_type=jax.ShapeDtypeStruct((e, H), F32),
        mesh=_mesh(),
        scratch_types=scratch,
        compiler_params=_sc_params(),
        name=name,
    )(*inputs)


def _edge_init(base, table, aidx, bidx):
    if base is not None:
        def cmb(gb, lb, i, sl):
            lb[0][i, sl] = lb[0][i, sl] + gb[0][i, sl] + gb[1][i, sl]
        return _edge_map([(table, aidx), (table, bidx)], [base], cmb, "l",
                         "sc_edge_init")

    def cmb(gb, lb, i, sl):
        gb[0][i, sl] = gb[0][i, sl] + gb[1][i, sl]
    return _edge_map([(table, aidx), (table, bidx)], [], cmb, "g",
                     "sc_edge_init")


def _gmix(tab0, x, e0, e1, inv):
    def cmb(gb, lb, i, sl):
        lb[0][i, sl] = gb[0][i, sl] * lb[0][i, sl] + gb[1][i, sl] * gb[2][i, sl]
    return _edge_map([(tab0, e0), (tab0, e1), (x, inv)], [x], cmb, "l",
                     "sc_gmix")


def _sym(x, inv):
    def cmb(gb, lb, i, sl):
        lb[0][i, sl] = 0.5 * (lb[0][i, sl] + gb[0][i, sl])
    return _edge_map([(x, inv)], [x], cmb, "l", "sc_sym")


def _tri_agg(table, a_s, b_s, t_s, offs, nout, rng, c2, pair):
    nr = nout // rng
    nmine = nr // NW
    assert nr % NW == 0 and nmine % 2 == 0
    offpad = offs.shape[0]
    reg = rng + 8

    idx_t = lambda: pltpu.VMEM((c2,), I32)
    g_t = lambda: pltpu.VMEM((c2, H), F32)
    if pair:
        scratch = [pltpu.VMEM((offpad,), I32), pltpu.VMEM((rng, H), F32),
                   idx_t(),
                   idx_t(), idx_t(), idx_t(),
                   idx_t(), idx_t(), idx_t(),
                   idx_t(), idx_t(), idx_t(),
                   g_t(), g_t(), g_t(), g_t(),
                   pltpu.VMEM_SHARED((16 * reg, H), F32),
                   pltpu.SemaphoreType.DMA, pltpu.SemaphoreType.DMA,
                   pltpu.SemaphoreType.DMA, pltpu.SemaphoreType.DMA]
    else:
        scratch = [pltpu.VMEM((offpad,), I32), pltpu.VMEM((rng, H), F32),
                   idx_t(),
                   idx_t(), idx_t(),
                   idx_t(), idx_t(),
                   idx_t(), idx_t(),
                   g_t(), g_t(),
                   pltpu.VMEM_SHARED((16 * reg, H), F32),
                   pltpu.SemaphoreType.DMA, pltpu.SemaphoreType.DMA,
                   pltpu.SemaphoreType.DMA, pltpu.SemaphoreType.DMA]

    def body(*refs):
        if pair:
            (tab_h, a_h, b_h, t_h, off_h, o_h,
             offv, zb, tlv, avA, tvA, bvA, avB, tvB, bvB, av2, tv2, bv2,
             g1A, g2A, g1B, g2B, acc, siA, siB, sgA, sgB) = refs
        else:
            (tab_h, a_h, t_h, off_h, o_h,
             offv, zb, tlv, avA, tvA, avB, tvB, av2, tv2,
             g1A, g1B, acc, siA, siB, sgA, sgB) = refs
            bvA = bvB = bv2 = g2A = g2B = None
        w = _wid()
        sid = lax.axis_index("s")
        base = sid * reg
        pltpu.sync_copy(off_h, offv)

        @plsc.parallel_loop(0, rng, 1, unroll=4)
        def _(i):
            for k in range(H // LANES):
                zb[i, pl.ds(k * LANES, LANES)] = jnp.zeros((LANES,), F32)

        pltpu.sync_copy(zb, acc.at[pl.ds(base, rng)])

        def startof(k):
            return (_sload(offv, w + k * NW) // 8) * 8

        def issue_idx(k, av, tv, bv, sem):
            st = startof(k)
            pltpu.async_copy(a_h.at[pl.ds(st, c2)], av, sem)
            pltpu.async_copy(t_h.at[pl.ds(st, c2)], tv, sem)
            if pair:
                pltpu.async_copy(b_h.at[pl.ds(st, c2)], bv, sem)

        def wait_idx(av, tv, bv, sem):
            pltpu.make_async_copy(a_h.at[pl.ds(0, c2)], av, sem).wait()
            pltpu.make_async_copy(t_h.at[pl.ds(0, c2)], tv, sem).wait()
            if pair:
                pltpu.make_async_copy(b_h.at[pl.ds(0, c2)], bv, sem).wait()

        def issue_gather(av, bv, g1, g2, sem):
            pltpu.async_copy(tab_h.at[av], g1, sem)
            if pair:
                pltpu.async_copy(tab_h.at[bv], g2, sem)

        def wait_gather(av, bv, g1, g2, sem):
            pltpu.make_async_copy(tab_h.at[av], g1, sem).wait()
            if pair:
                pltpu.make_async_copy(tab_h.at[bv], g2, sem).wait()

        def accumulate(tv_, g1_, g2_, lo):
            @plsc.parallel_loop(0, c2 // LANES, 1, unroll=4)
            def _(gi):
                sl = pl.ds(gi * LANES, LANES)
                t16 = tv_[sl]
                inr = (t16 >= lo) & (t16 < lo + rng)
                tlv[sl] = jnp.where(inr, t16 - lo + base, base + rng)

            if pair:
                @plsc.parallel_loop(0, c2, 1, unroll=4)
                def _(i):
                    for k in range(H // LANES):
                        sl = pl.ds(k * LANES, LANES)
                        g1_[i, sl] = g1_[i, sl] * g2_[i, sl]

            pltpu.sync_copy(g1_, acc.at[tlv], add=True)

        def process(k, tvP, g1P, g2P):
            r = w + k * NW
            o0 = _sload(offv, r)
            o1 = _sload(offv, r + 1)
            start = (o0 // 8) * 8
            lo = r * rng
            accumulate(tvP, g1P, g2P, lo)
            nch = (o1 - start + c2 - 1) // c2

            @pl.loop(1, nch)
            def _(ci):
                st = start + ci * c2
                pltpu.sync_copy(a_h.at[pl.ds(st, c2)], av2)
                pltpu.sync_copy(t_h.at[pl.ds(st, c2)], tv2)
                if pair:
                    pltpu.sync_copy(b_h.at[pl.ds(st, c2)], bv2)
                pltpu.sync_copy(tab_h.at[av2], g1P)
                if pair:
                    pltpu.sync_copy(tab_h.at[bv2], g2P)
                accumulate(tv2, g1P, g2P, lo)

            pltpu.sync_copy(acc.at[pl.ds(base, rng)], o_h.at[pl.ds(lo, rng)])
            pltpu.sync_copy(zb, acc.at[pl.ds(base, rng)])

        issue_idx(0, avA, tvA, bvA, siA)
        wait_idx(avA, tvA, bvA, siA)
        issue_gather(avA, bvA, g1A, g2A, sgA)
        issue_idx(1, avB, tvB, bvB, siB)

        def phase(k, avP, tvP, bvP, g1P, g2P, siP, sgP,
                  avQ, tvQ, bvQ, g1Q, g2Q, siQ, sgQ):
            wait_gather(avP, bvP, g1P, g2P, sgP)

            @pl.when(k + 1 < nmine)
            def _():
                wait_idx(avQ, tvQ, bvQ, siQ)
                issue_gather(avQ, bvQ, g1Q, g2Q, sgQ)

                @pl.when(k + 2 < nmine)
                def _():
                    issue_idx(k + 2, avP, tvP, bvP, siP)

            process(k, tvP, g1P, g2P)

        @pl.loop(0, nmine // 2)
        def _(k2):
            k = 2 * k2
            phase(k, avA, tvA, bvA, g1A, g2A, siA, sgA,
                  avB, tvB, bvB, g1B, g2B, siB, sgB)
            phase(k + 1, avB, tvB, bvB, g1B, g2B, siB, sgB,
                  avA, tvA, bvA, g1A, g2A, siA, sgA)

    kern = pl.kernel(
        body,
        out_type=jax.ShapeDtypeStruct((nout, H), F32),
        mesh=_mesh(),
        scratch_types=scratch,
        compiler_params=_sc_params(),
        name="sc_tri_agg" if pair else "sc_seg_add",
    )
    if pair:
        return kern(table, a_s, b_s, t_s, offs)
    return kern(table, a_s, t_s, offs)


def _mkoffs(t_s, nout, rng):
    nr = nout // rng
    offs = jnp.searchsorted(
        t_s, (jnp.arange(nr + 1, dtype=I32) * rng).astype(I32)
    ).astype(I32)
    offpad = ((nr + 1 + 15) // 16) * 16
    return jnp.pad(offs, (0, offpad - nr - 1), constant_values=t_s.shape[0])


def _sort3(tgt, a, b, nout, rng, c2):
    t_s, a_s, b_s = lax.sort([tgt, a, b], num_keys=1)
    offs = _mkoffs(t_s, nout, rng)
    pad = c2 + 8
    t_s = jnp.pad(t_s, (0, pad), constant_values=1 << 30)
    a_s = jnp.pad(a_s, (0, pad))
    b_s = jnp.pad(b_s, (0, pad))
    return t_s, a_s, b_s, offs


def _sort2(tgt, a, nout, rng, c2):
    t_s, a_s = lax.sort([tgt, a], num_keys=1)
    offs = _mkoffs(t_s, nout, rng)
    pad = c2 + 8
    t_s = jnp.pad(t_s, (0, pad), constant_values=1 << 30)
    a_s = jnp.pad(a_s, (0, pad))
    return t_s, a_s, offs


def kernel(x, edge_attr, edge_index, edge_index2, triangle_1_1_1,
           triangle_1_1_2, triangle_2_2_1, triangle_2_2_2, inverse_edge_1,
           inverse_edge_2, num_nodes, batch0, W0, b0, W1, b1, W2, b2, Wk, Wp,
           Wm1, bm1, Wm2, bm2):
    n = x.shape[0]
    e1 = edge_index.shape[1]
    e2 = edge_index2.shape[1]
    nl = Wk.shape[0]
    npad = 10240

    i32 = lambda v: v.astype(I32)
    ei0, ei1 = i32(edge_index[0]), i32(edge_index[1])
    e20, e21 = i32(edge_index2[0]), i32(edge_index2[1])
    inv1, inv2 = i32(inverse_edge_1), i32(inverse_edge_2)

    RNG, C2 = 40, 128
    s111 = _sort3(i32(triangle_1_1_1[0]), i32(triangle_1_1_1[1]),
                  i32(triangle_1_1_1[2]), e1, RNG, C2)
    s112 = _sort3(i32(triangle_1_1_2[2]), i32(triangle_1_1_2[0]),
                  i32(triangle_1_1_2[1]), e2, RNG, C2)
    s221 = _sort3(i32(triangle_2_2_1[2]), i32(triangle_2_2_1[0]),
                  i32(triangle_2_2_1[1]), e1, RNG, C2)
    s222 = _sort3(i32(triangle_2_2_2[2]), i32(triangle_2_2_2[0]),
                  i32(triangle_2_2_2[1]), e2, RNG, C2)

    NRNG, NC2 = 32, 256
    ar1 = jnp.arange(e1, dtype=I32)
    n1s = _sort2(jnp.concatenate([ei0, ei1]), jnp.concatenate([ar1, ar1]),
                 npad, NRNG, NC2)
    ar2 = jnp.arange(e2, dtype=I32)
    n2s = _sort2(jnp.concatenate([e20, e21]), jnp.concatenate([ar2, ar2]),
                 npad, NRNG, NC2)

    xpad = jnp.zeros((npad, 16), F32).at[:n, :11].set(x)
    wcat = jnp.zeros((16, 3 * H), F32).at[:11].set(
        jnp.concatenate([W0, W1[5:], W2], axis=1)
    )
    bcat = jnp.concatenate([b0, jnp.zeros((H,), F32), 0.5 * b2])[None]
    xt = _mm(xpad, wcat, bcat, rb=2048)
    attr0 = xt[:, :H]
    xw1h = xt[:, H:2 * H]
    xw2b = xt[:, 2 * H:]

    epad = jnp.zeros((e1, 8), F32).at[:, :5].set(edge_attr)
    w1a = jnp.zeros((8, H), F32).at[:5].set(W1[:5])
    edgepart = _mm(epad, w1a, b1[None], rb=4000)

    attr1 = _edge_init(edgepart, xw1h, ei0, ei1)
    attr2 = _edge_init(None, xw2b, e20, e21)

    for l in range(nl):
        agg111 = _tri_agg(attr1, s111[1], s111[2], s111[0], s111[3], e1, RNG, C2, True)
        agg112 = _tri_agg(attr1, s112[1], s112[2], s112[0], s112[3], e2, RNG, C2, True)
        agg221 = _tri_agg(attr2, s221[1], s221[2], s221[0], s221[3], e1, RNG, C2, True)
        agg222 = _tri_agg(attr2, s222[1], s222[2], s222[0], s222[3], e2, RNG, C2, True)
        agg011 = _gmix(attr0, attr1, ei0, ei1, inv1)
        agg022 = _gmix(attr0, attr2, e20, e21, inv2)
        attr1 = _upd(attr1, agg111, agg011, agg221, Wk[l, 0], Wk[l, 1], Wk[l, 4])
        attr2 = _upd(attr2, agg022, agg112, agg222, Wk[l, 2], Wk[l, 3], Wk[l, 5])
        attr1 = _sym(attr1, inv1)
        attr2 = _sym(attr2, inv2)

    na1 = _tri_agg(attr1, n1s[1], None, n1s[0], n1s[2], npad, NRNG, NC2, False)
    na2 = _tri_agg(attr2, n2s[1], None, n2s[0], n2s[2], npad, NRNG, NC2, False)

    bpad = jnp.full((npad,), 600, I32).at[:n].set(i32(batch0))
    batch_r = bpad.reshape(npad // 128, 1, 128)
    wm1p = jnp.zeros((H, H), F32).at[:, :Wm1.shape[1]].set(Wm1)
    bm1p = jnp.zeros((1, H), F32).at[0, :bm1.shape[0]].set(bm1)
    wm2p = jnp.zeros((H, H), F32).at[:Wm2.shape[0], 0].set(Wm2[:, 0])
    bm2p = jnp.zeros((1, H), F32) + bm2[0]
    out = _head(attr0, na1, na2, batch_r, Wp, wm1p, bm1p, wm2p, bm2p)
    return out
